# baseline (speedup 1.0000x reference)
import jax
import jax.numpy as jnp
from jax.experimental import pallas as pl
from jax.experimental.pallas import tpu as pltpu

N = 10000
E = 2000
F_IN = 256
HID = 256
F_OUT = 256

BN1 = 1000
BN3 = 1000
INV_SCALE = 1.0 / 16.0

_NT = (((1,), (1,)), ((), ()))
_TN = (((0,), (0,)), ((), ()))
_NN = (((1,), (0,)), ((), ()))


def _dot(a, b, dims=_NN):
    return jax.lax.dot_general(a, b, dims, preferred_element_type=jnp.float32)


def _he_kernel(h_ref, x_ref, o_ref):
    i = pl.program_id(0)

    @pl.when(i == 0)
    def _():
        o_ref[...] = jnp.zeros_like(o_ref)

    o_ref[...] += _dot(h_ref[...], x_ref[...], _TN)


def _node_kernel(he_ref, wq_ref, bq_ref, wk_ref, bk_ref, wv_ref, bv_ref,
                 wnk_ref, bnk_ref, x_ref, h_ref, wnq_ref, bnq_ref,
                 wt_ref, bt_ref, o_ref, att_ref, knt_ref):
    i = pl.program_id(0)

    @pl.when(i == 0)
    def _():
        he = he_ref[...]
        q = _dot(he, wq_ref[...]) + bq_ref[...]
        k = _dot(he, wk_ref[...]) + bk_ref[...]
        v = _dot(he, wv_ref[...]) + bv_ref[...]
        s = _dot(q, k, _NT) * INV_SCALE
        s = s - jnp.max(s, axis=-1, keepdims=True)
        p = jnp.exp(s)
        p = p / jnp.sum(p, axis=-1, keepdims=True)
        att = _dot(p, v)
        att_ref[...] = att.astype(jnp.bfloat16)
        kn = _dot(att, wnk_ref[...]) + bnk_ref[...]
        knt_ref[...] = kn.T

    qn = _dot(x_ref[...], wnq_ref[...]) + bnq_ref[...]
    s = _dot(qn, knt_ref[...]) * INV_SCALE
    s = s - jnp.max(s, axis=-1, keepdims=True)
    p = jnp.exp(s)
    p = p / jnp.sum(p, axis=-1, keepdims=True)
    hatt = (h_ref[...].astype(jnp.float32) * p).astype(jnp.bfloat16)
    agg = _dot(hatt, att_ref[...])
    o_ref[...] = _dot(agg, wt_ref[...]) + bt_ref[...]


def kernel(X, H_norm, Wq, bq, Wk, bk, Wv, bv, Wnq, bnq, Wnk, bnk, Wt, bt):
    bq2 = bq.reshape(1, HID)
    bk2 = bk.reshape(1, HID)
    bv2 = bv.reshape(1, HID)
    bnq2 = bnq.reshape(1, HID)
    bnk2 = bnk.reshape(1, HID)
    bt2 = bt.reshape(1, F_OUT)

    he = pl.pallas_call(
        _he_kernel,
        grid=(N // BN1,),
        in_specs=[
            pl.BlockSpec((BN1, E), lambda i: (i, 0)),
            pl.BlockSpec((BN1, F_IN), lambda i: (i, 0)),
        ],
        out_specs=pl.BlockSpec((E, F_IN), lambda i: (0, 0)),
        out_shape=jax.ShapeDtypeStruct((E, F_IN), jnp.float32),
        compiler_params=pltpu.CompilerParams(
            dimension_semantics=("arbitrary",)),
    )(H_norm, X)

    wfull = lambda shape: pl.BlockSpec(shape, lambda i: (0, 0))
    out = pl.pallas_call(
        _node_kernel,
        grid=(N // BN3,),
        in_specs=[
            wfull((E, F_IN)),
            wfull((F_IN, HID)), wfull((1, HID)),
            wfull((F_IN, HID)), wfull((1, HID)),
            wfull((F_IN, HID)), wfull((1, HID)),
            wfull((HID, HID)), wfull((1, HID)),
            pl.BlockSpec((BN3, F_IN), lambda i: (i, 0)),
            pl.BlockSpec((BN3, E), lambda i: (i, 0)),
            wfull((F_IN, HID)), wfull((1, HID)),
            wfull((HID, F_OUT)), wfull((1, F_OUT)),
        ],
        out_specs=pl.BlockSpec((BN3, F_OUT), lambda i: (i, 0)),
        out_shape=jax.ShapeDtypeStruct((N, F_OUT), jnp.float32),
        scratch_shapes=[
            pltpu.VMEM((E, HID), jnp.bfloat16),
            pltpu.VMEM((HID, E), jnp.float32),
        ],
        compiler_params=pltpu.CompilerParams(
            dimension_semantics=("arbitrary",)),
    )(he, Wq, bq2, Wk, bk2, Wv, bv2, Wnk, bnk2, X, H_norm.astype(jnp.bfloat16), Wnq, bnq2, Wt, bt2)

    return out


# speedup vs baseline: 2.3186x; 2.3186x over previous
import jax
import jax.numpy as jnp
from jax.experimental import pallas as pl
from jax.experimental.pallas import tpu as pltpu

N = 10000
E = 2000
F_IN = 256
HID = 256
F_OUT = 256

BE1 = 400
BN3 = 1024
INV_SCALE = 1.0 / 16.0

_NT = (((1,), (1,)), ((), ()))
_NN = (((1,), (0,)), ((), ()))


def _dot(a, b, dims=_NN):
    return jax.lax.dot_general(a, b, dims, preferred_element_type=jnp.float32)


def _he_kernel(ht_ref, x_ref, o_ref):
    o_ref[...] = _dot(ht_ref[...], x_ref[...])


def _node_kernel(he_ref, wq_ref, bq_ref, wk_ref, bk_ref, wv_ref, bv_ref,
                 wnk_ref, bnk_ref, x_ref, ht_ref, wnq_ref, bnq_ref,
                 wt_ref, bt_ref, o_ref, att_ref, knt_ref):
    i = pl.program_id(0)

    @pl.when(i == 0)
    def _():
        he = he_ref[...]
        q = _dot(he, wq_ref[...]) + bq_ref[...]
        k = _dot(he, wk_ref[...]) + bk_ref[...]
        v = _dot(he, wv_ref[...]) + bv_ref[...]
        s = _dot(q, k, _NT) * INV_SCALE
        s = s - jnp.max(s, axis=-1, keepdims=True)
        p = jnp.exp(s)
        p = p / jnp.sum(p, axis=-1, keepdims=True)
        att = _dot(p, v)
        att_ref[...] = att.astype(jnp.bfloat16)
        kn = _dot(att, wnk_ref[...]) + bnk_ref[...]
        knt_ref[...] = kn.T

    qn = _dot(x_ref[...], wnq_ref[...]) + bnq_ref[...]
    s = _dot(qn, knt_ref[...]) * INV_SCALE
    s = s - jnp.max(s, axis=-1, keepdims=True)
    p = jnp.exp(s)
    p = p / jnp.sum(p, axis=-1, keepdims=True)
    h_blk = ht_ref[...].T
    hatt = (h_blk * p).astype(jnp.bfloat16)
    agg = _dot(hatt, att_ref[...])
    o_ref[...] = _dot(agg, wt_ref[...]) + bt_ref[...]


def kernel(X, H_norm, Wq, bq, Wk, bk, Wv, bv, Wnq, bnq, Wnk, bnk, Wt, bt):
    bq2 = bq.reshape(1, HID)
    bk2 = bk.reshape(1, HID)
    bv2 = bv.reshape(1, HID)
    bnq2 = bnq.reshape(1, HID)
    bnk2 = bnk.reshape(1, HID)
    bt2 = bt.reshape(1, F_OUT)
    HT = H_norm.T

    he = pl.pallas_call(
        _he_kernel,
        grid=(E // BE1,),
        in_specs=[
            pl.BlockSpec((BE1, N), lambda i: (i, 0)),
            pl.BlockSpec((N, F_IN), lambda i: (0, 0)),
        ],
        out_specs=pl.BlockSpec((BE1, F_IN), lambda i: (i, 0)),
        out_shape=jax.ShapeDtypeStruct((E, F_IN), jnp.float32),
        compiler_params=pltpu.CompilerParams(
            dimension_semantics=("arbitrary",)),
    )(HT, X)

    wfull = lambda shape: pl.BlockSpec(shape, lambda i: (0, 0))
    out = pl.pallas_call(
        _node_kernel,
        grid=(pl.cdiv(N, BN3),),
        in_specs=[
            wfull((E, F_IN)),
            wfull((F_IN, HID)), wfull((1, HID)),
            wfull((F_IN, HID)), wfull((1, HID)),
            wfull((F_IN, HID)), wfull((1, HID)),
            wfull((HID, HID)), wfull((1, HID)),
            pl.BlockSpec((BN3, F_IN), lambda i: (i, 0)),
            pl.BlockSpec((E, BN3), lambda i: (0, i)),
            wfull((F_IN, HID)), wfull((1, HID)),
            wfull((HID, F_OUT)), wfull((1, F_OUT)),
        ],
        out_specs=pl.BlockSpec((BN3, F_OUT), lambda i: (i, 0)),
        out_shape=jax.ShapeDtypeStruct((N, F_OUT), jnp.float32),
        scratch_shapes=[
            pltpu.VMEM((E, HID), jnp.bfloat16),
            pltpu.VMEM((HID, E), jnp.float32),
        ],
        compiler_params=pltpu.CompilerParams(
            dimension_semantics=("arbitrary",)),
    )(he, Wq, bq2, Wk, bk2, Wv, bv2, Wnk, bnk2, X, HT, Wnq, bnq2, Wt, bt2)

    return out


# speedup vs baseline: 2.6716x; 1.1523x over previous
import jax
import jax.numpy as jnp
from jax.experimental import pallas as pl
from jax.experimental.pallas import tpu as pltpu

N = 10000
E = 2000
F_IN = 256
HID = 256
F_OUT = 256

BE1 = 400
BN3 = 1024
INV_SCALE = 1.0 / 16.0

_NT = (((1,), (1,)), ((), ()))
_NN = (((1,), (0,)), ((), ()))


def _dot(a, b, dims=_NN):
    return jax.lax.dot_general(a, b, dims, preferred_element_type=jnp.float32)


def _he_kernel(ht_ref, x_ref, o_ref):
    o_ref[...] = _dot(ht_ref[...], x_ref[...])


def _node_kernel(he_ref, wq_ref, bq_ref, wk_ref, bk_ref, wv_ref, bv_ref,
                 wnk_ref, bnk_ref, x_ref, ht_ref, wnq_ref, bnq_ref,
                 wt_ref, bt_ref, o_ref, att_ref, kn_ref):
    i = pl.program_id(0)

    @pl.when(i == 0)
    def _():
        he = he_ref[...]
        q = _dot(he, wq_ref[...]) + bq_ref[...]
        k = _dot(he, wk_ref[...]) + bk_ref[...]
        v = _dot(he, wv_ref[...]) + bv_ref[...]
        s = _dot(q, k, _NT) * INV_SCALE
        s = s - jnp.max(s, axis=-1, keepdims=True)
        p = jnp.exp(s)
        p = p / jnp.sum(p, axis=-1, keepdims=True)
        att = _dot(p, v)
        att_ref[...] = att.T.astype(jnp.bfloat16)
        kn = _dot(att, wnk_ref[...]) + bnk_ref[...]
        kn_ref[...] = kn

    qn = _dot(x_ref[...], wnq_ref[...]) + bnq_ref[...]
    st = _dot(kn_ref[...], qn.T) * INV_SCALE
    st = st - jnp.max(st, axis=0, keepdims=True)
    pt = jnp.exp(st)
    pt = pt / jnp.sum(pt, axis=0, keepdims=True)
    hatt_t = (ht_ref[...] * pt).astype(jnp.bfloat16)
    agg_t = _dot(att_ref[...], hatt_t)
    o_ref[...] = _dot(agg_t.T, wt_ref[...]) + bt_ref[...]


def kernel(X, H_norm, Wq, bq, Wk, bk, Wv, bv, Wnq, bnq, Wnk, bnk, Wt, bt):
    bq2 = bq.reshape(1, HID)
    bk2 = bk.reshape(1, HID)
    bv2 = bv.reshape(1, HID)
    bnq2 = bnq.reshape(1, HID)
    bnk2 = bnk.reshape(1, HID)
    bt2 = bt.reshape(1, F_OUT)
    HT = H_norm.T

    he = pl.pallas_call(
        _he_kernel,
        grid=(E // BE1,),
        in_specs=[
            pl.BlockSpec((BE1, N), lambda i: (i, 0)),
            pl.BlockSpec((N, F_IN), lambda i: (0, 0)),
        ],
        out_specs=pl.BlockSpec((BE1, F_IN), lambda i: (i, 0)),
        out_shape=jax.ShapeDtypeStruct((E, F_IN), jnp.float32),
        compiler_params=pltpu.CompilerParams(
            dimension_semantics=("arbitrary",)),
    )(HT, X)

    wfull = lambda shape: pl.BlockSpec(shape, lambda i: (0, 0))
    out = pl.pallas_call(
        _node_kernel,
        grid=(pl.cdiv(N, BN3),),
        in_specs=[
            wfull((E, F_IN)),
            wfull((F_IN, HID)), wfull((1, HID)),
            wfull((F_IN, HID)), wfull((1, HID)),
            wfull((F_IN, HID)), wfull((1, HID)),
            wfull((HID, HID)), wfull((1, HID)),
            pl.BlockSpec((BN3, F_IN), lambda i: (i, 0)),
            pl.BlockSpec((E, BN3), lambda i: (0, i)),
            wfull((F_IN, HID)), wfull((1, HID)),
            wfull((HID, F_OUT)), wfull((1, F_OUT)),
        ],
        out_specs=pl.BlockSpec((BN3, F_OUT), lambda i: (i, 0)),
        out_shape=jax.ShapeDtypeStruct((N, F_OUT), jnp.float32),
        scratch_shapes=[
            pltpu.VMEM((HID, E), jnp.bfloat16),
            pltpu.VMEM((E, HID), jnp.float32),
        ],
        compiler_params=pltpu.CompilerParams(
            dimension_semantics=("arbitrary",)),
    )(he, Wq, bq2, Wk, bk2, Wv, bv2, Wnk, bnk2, X, HT, Wnq, bnq2, Wt, bt2)

    return out
